# baseline (speedup 1.0000x reference)
import functools

import jax
import jax.numpy as jnp
from jax import lax
from jax.experimental import pallas as pl
from jax.experimental.pallas import tpu as pltpu
from jax.experimental.pallas import tpu_sc as plsc

_NC = 2
_NS = 16
_NW = _NC * _NS
_L = 16


def _make_sc_prep(E, N):
    ept_h = E // _NS
    epw = E // _NW
    npad = ((N + _NS * _L - 1) // (_NS * _L)) * (_NS * _L)
    nslice = npad // _NS
    g_iters = (epw + _L - 1) // _L
    tail_base = (g_iters - 1) * _L
    tail_valid = epw - tail_base
    g_main = (g_iters - 1) // 4 * 4

    mesh = plsc.VectorSubcoreMesh(core_axis_name="c", subcore_axis_name="s")

    @functools.partial(
        pl.kernel,
        out_type=(
            jax.ShapeDtypeStruct((E,), jnp.int32),
            jax.ShapeDtypeStruct((E,), jnp.float32),
        ),
        mesh=mesh,
        compiler_params=pltpu.CompilerParams(needs_layout_passes=False),
        scratch_types=[
            pltpu.VMEM((ept_h,), jnp.int32),
            pltpu.VMEM((g_iters * _L,), jnp.int32),
            pltpu.VMEM((npad,), jnp.float32),
            pltpu.VMEM((_NS, nslice), jnp.float32),
            pltpu.VMEM((nslice,), jnp.float32),
            pltpu.VMEM((npad,), jnp.float32),
            pltpu.VMEM((N,), jnp.int32),
            pltpu.VMEM((g_iters * _L,), jnp.int32),
            pltpu.VMEM((g_iters * _L,), jnp.float32),
            pltpu.VMEM_SHARED((_NS, npad), jnp.float32),
            pltpu.VMEM_SHARED((npad,), jnp.float32),
            pltpu.SemaphoreType.DMA,
            pltpu.SemaphoreType.DMA,
        ],
    )
    def sc_prep(esrc_hbm, batch_hbm, ge_hbm, we_hbm,
                ebuf, ebuf_c, hist, parts, winv_s, winv, batch_l, geb, web,
                sh_hist, sh_winv, sem_b, sem_e):
        c = lax.axis_index("c")
        s = lax.axis_index("s")
        w = c * _NS + s

        cp_batch = pltpu.async_copy(batch_hbm, batch_l, sem_b)
        cp_edges = pltpu.async_copy(esrc_hbm.at[pl.ds(w * epw, epw)],
                                    ebuf_c.at[pl.ds(0, epw)], sem_e)

        @plsc.parallel_loop(0, npad // _L)
        def _(i):
            hist[pl.ds(i * _L, _L)] = jnp.zeros((_L,), jnp.float32)

        pltpu.sync_copy(esrc_hbm.at[pl.ds(s * ept_h, ept_h)], ebuf)
        ones = jnp.ones((_L,), jnp.float32)

        def hist_body(i, _):
            base = i * (5 * _L)
            for k in range(5):
                idx = ebuf[pl.ds(base + k * _L, _L)]
                plsc.addupdate_scatter(hist, [idx], ones)
            return 0
        lax.fori_loop(0, ept_h // (5 * _L), hist_body, 0)

        pltpu.sync_copy(hist, sh_hist.at[s])
        plsc.subcore_barrier()

        pltpu.sync_copy(sh_hist.at[:, pl.ds(s * nslice, nslice)], parts)

        @plsc.parallel_loop(0, nslice // _L)
        def _(j):
            acc = jnp.zeros((_L,), jnp.float32)
            for t in range(_NS):
                acc = acc + parts[t, pl.ds(j * _L, _L)]
            winv_s[pl.ds(j * _L, _L)] = 1.0 / jnp.maximum(acc, 1.0)

        pltpu.sync_copy(winv_s, sh_winv.at[pl.ds(s * nslice, nslice)])
        plsc.subcore_barrier()

        pltpu.sync_copy(sh_winv, winv)
        cp_batch.wait()
        cp_edges.wait()
        lane = lax.iota(jnp.int32, _L)
        tail = ebuf_c[pl.ds(tail_base, _L)]
        ebuf_c[pl.ds(tail_base, _L)] = jnp.where(lane < tail_valid, tail, 0)

        @plsc.parallel_loop(0, g_main // 4, unroll=4)
        def _(i4):
            for k in range(4):
                off = (i4 * 4 + k) * _L
                idx = ebuf_c[pl.ds(off, _L)]
                geb[pl.ds(off, _L)] = plsc.load_gather(batch_l, [idx])
                web[pl.ds(off, _L)] = plsc.load_gather(winv, [idx])

        @plsc.parallel_loop(g_main, g_iters)
        def _(i):
            idx = ebuf_c[pl.ds(i * _L, _L)]
            geb[pl.ds(i * _L, _L)] = plsc.load_gather(batch_l, [idx])
            web[pl.ds(i * _L, _L)] = plsc.load_gather(winv, [idx])

        pltpu.sync_copy(geb.at[pl.ds(0, epw)], ge_hbm.at[pl.ds(w * epw, epw)])
        pltpu.sync_copy(web.at[pl.ds(0, epw)], we_hbm.at[pl.ds(w * epw, epw)])

    return sc_prep


def _mm(a, b):
    return lax.dot_general(a, b, (((1,), (0,)), ((), ())),
                           preferred_element_type=jnp.float32,
                           precision=lax.Precision.HIGHEST)


def _mm_fast(a, b):
    return lax.dot_general(a, b, (((1,), (0,)), ((), ())),
                           preferred_element_type=jnp.float32)


def _make_tc_nodeagg(N, B, DIM, n_blk, nb):
    def body(bt_ref, x_ref, xsum_ref, cnt_ref, acc_v, cnt):
        i = pl.program_id(0)

        @pl.when(i == 0)
        def _():
            acc_v[...] = jnp.zeros_like(acc_v)
            cnt[...] = jnp.zeros_like(cnt)

        bt = bt_ref[0]
        niota = lax.broadcasted_iota(jnp.int32, (B, nb), 0)
        onehot_v = jnp.where(bt == niota, 1.0, 0.0)
        acc_v[...] = acc_v[...] + _mm_fast(onehot_v.astype(jnp.bfloat16),
                                           x_ref[...].astype(jnp.bfloat16))
        cnt[...] = cnt[...] + jnp.sum(onehot_v, axis=1, keepdims=True)

        @pl.when(i == n_blk - 1)
        def _():
            xsum_ref[...] = acc_v[...]
            cnt_ref[...] = cnt[...]

    full2 = lambda i: (0, 0)
    return pl.pallas_call(
        body,
        grid=(n_blk,),
        in_specs=[
            pl.BlockSpec((1, 1, nb), lambda i: (i, 0, 0)),
            pl.BlockSpec((nb, DIM), lambda i: (i, 0)),
        ],
        out_specs=[
            pl.BlockSpec((B, DIM), full2),
            pl.BlockSpec((B, 128), full2),
        ],
        out_shape=[
            jax.ShapeDtypeStruct((B, DIM), jnp.float32),
            jax.ShapeDtypeStruct((B, 128), jnp.float32),
        ],
        scratch_shapes=[
            pltpu.VMEM((B, DIM), jnp.float32),
            pltpu.VMEM((B, 128), jnp.float32),
        ],
        compiler_params=pltpu.CompilerParams(
            dimension_semantics=("arbitrary",)),
    )


def _make_tc_edge(E, B, DIM, n_blk, eb):
    def body(ge_ref, we_ref, ea_ref, xsum_ref, cnt_ref, u_ref,
             w1a_ref, w1b_ref, w1c_ref, b1_ref, w2_ref, b2_ref,
             out_ref, acc_e):
        i = pl.program_id(0)

        @pl.when(i == 0)
        def _():
            acc_e[...] = jnp.zeros_like(acc_e)

        ge = ge_ref[0]
        we = we_ref[0]
        giota = lax.broadcasted_iota(jnp.int32, (B, eb), 0)
        onehot_e = jnp.where(ge == giota, jnp.broadcast_to(we, (B, eb)), 0.0)
        acc_e[...] = acc_e[...] + _mm_fast(onehot_e.astype(jnp.bfloat16),
                                           ea_ref[...].astype(jnp.bfloat16))

        @pl.when(i == n_blk - 1)
        def _():
            n = jnp.maximum(cnt_ref[:, 0:1], 1.0)
            ue = acc_e[...] / n
            uv = xsum_ref[...] / n
            h = (_mm(ue, w1a_ref[...]) + _mm(uv, w1b_ref[...])
                 + _mm(u_ref[...], w1c_ref[...]) + b1_ref[...])
            h = jnp.maximum(h, 0.0)
            out_ref[...] = _mm(h, w2_ref[...]) + b2_ref[...]

    full2 = lambda i: (0, 0)
    return pl.pallas_call(
        body,
        grid=(n_blk,),
        in_specs=[
            pl.BlockSpec((1, 1, eb), lambda i: (i, 0, 0)),
            pl.BlockSpec((1, 1, eb), lambda i: (i, 0, 0)),
            pl.BlockSpec((eb, DIM), lambda i: (i, 0)),
            pl.BlockSpec((B, DIM), full2),
            pl.BlockSpec((B, 128), full2),
            pl.BlockSpec((B, DIM), full2),
            pl.BlockSpec((DIM, DIM), full2),
            pl.BlockSpec((DIM, DIM), full2),
            pl.BlockSpec((DIM, DIM), full2),
            pl.BlockSpec((1, DIM), full2),
            pl.BlockSpec((DIM, DIM), full2),
            pl.BlockSpec((1, DIM), full2),
        ],
        out_specs=pl.BlockSpec((B, DIM), full2),
        out_shape=jax.ShapeDtypeStruct((B, DIM), jnp.float32),
        scratch_shapes=[
            pltpu.VMEM((B, DIM), jnp.float32),
        ],
        compiler_params=pltpu.CompilerParams(
            dimension_semantics=("arbitrary",)),
    )


def kernel(x, edge_index, edge_attr, u, batch, W1, b1, W2, b2):
    N, DIM = x.shape
    E = edge_attr.shape[0]
    B = u.shape[0]

    n_eblk = 40
    eb = E // n_eblk
    n_nblk = 10
    nb = N // n_nblk

    esrc = edge_index[0]
    ge, we = _make_sc_prep(E, N)(esrc, batch)
    xsum, cnt = _make_tc_nodeagg(N, B, DIM, n_nblk, nb)(
        batch.reshape(n_nblk, 1, nb), x)

    out = _make_tc_edge(E, B, DIM, n_eblk, eb)(
        ge.reshape(n_eblk, 1, eb), we.reshape(n_eblk, 1, eb), edge_attr,
        xsum, cnt, u,
        W1[0:DIM], W1[DIM:2 * DIM], W1[2 * DIM:3 * DIM],
        b1.reshape(1, DIM), W2, b2.reshape(1, DIM))
    return out


# speedup vs baseline: 1.0884x; 1.0884x over previous
import functools

import jax
import jax.numpy as jnp
from jax import lax
from jax.experimental import pallas as pl
from jax.experimental.pallas import tpu as pltpu
from jax.experimental.pallas import tpu_sc as plsc

_NC = 2
_NS = 16
_NW = _NC * _NS
_L = 16


def _make_sc_prep(E, N):
    ept_h = E // _NS
    epw = E // _NW
    npad = ((N + _NS * _L - 1) // (_NS * _L)) * (_NS * _L)
    nslice = npad // _NS
    g_iters = (epw + _L - 1) // _L
    tail_base = (g_iters - 1) * _L
    tail_valid = epw - tail_base
    g_main = (g_iters - 1) // 4 * 4

    mesh = plsc.VectorSubcoreMesh(core_axis_name="c", subcore_axis_name="s")

    @functools.partial(
        pl.kernel,
        out_type=(
            jax.ShapeDtypeStruct((E,), jnp.int32),
            jax.ShapeDtypeStruct((E,), jnp.float32),
        ),
        mesh=mesh,
        compiler_params=pltpu.CompilerParams(needs_layout_passes=False),
        scratch_types=[
            pltpu.VMEM((ept_h,), jnp.int32),
            pltpu.VMEM((g_iters * _L,), jnp.int32),
            pltpu.VMEM((npad,), jnp.float32),
            pltpu.VMEM((_NS, nslice), jnp.float32),
            pltpu.VMEM((nslice,), jnp.float32),
            pltpu.VMEM((npad,), jnp.float32),
            pltpu.VMEM((N,), jnp.int32),
            pltpu.VMEM((g_iters * _L,), jnp.int32),
            pltpu.VMEM((g_iters * _L,), jnp.float32),
            pltpu.VMEM_SHARED((_NS, npad), jnp.float32),
            pltpu.VMEM_SHARED((npad,), jnp.float32),
            pltpu.SemaphoreType.DMA,
            pltpu.SemaphoreType.DMA,
        ],
    )
    def sc_prep(esrc_hbm, batch_hbm, ge_hbm, we_hbm,
                ebuf, ebuf_c, hist, parts, winv_s, winv, batch_l, geb, web,
                sh_hist, sh_winv, sem_b, sem_e):
        c = lax.axis_index("c")
        s = lax.axis_index("s")
        w = c * _NS + s

        cp_batch = pltpu.async_copy(batch_hbm, batch_l, sem_b)
        cp_edges = pltpu.async_copy(esrc_hbm.at[pl.ds(w * epw, epw)],
                                    ebuf_c.at[pl.ds(0, epw)], sem_e)

        @plsc.parallel_loop(0, npad // _L)
        def _(i):
            hist[pl.ds(i * _L, _L)] = jnp.zeros((_L,), jnp.float32)

        pltpu.sync_copy(esrc_hbm.at[pl.ds(s * ept_h, ept_h)], ebuf)
        ones = jnp.ones((_L,), jnp.float32)

        def hist_body(i, _):
            base = i * (5 * _L)
            for k in range(5):
                idx = ebuf[pl.ds(base + k * _L, _L)]
                plsc.addupdate_scatter(hist, [idx], ones)
            return 0
        lax.fori_loop(0, ept_h // (5 * _L), hist_body, 0)

        pltpu.sync_copy(hist, sh_hist.at[s])
        plsc.subcore_barrier()

        pltpu.sync_copy(sh_hist.at[:, pl.ds(s * nslice, nslice)], parts)

        @plsc.parallel_loop(0, nslice // _L)
        def _(j):
            acc = jnp.zeros((_L,), jnp.float32)
            for t in range(_NS):
                acc = acc + parts[t, pl.ds(j * _L, _L)]
            winv_s[pl.ds(j * _L, _L)] = 1.0 / jnp.maximum(acc, 1.0)

        pltpu.sync_copy(winv_s, sh_winv.at[pl.ds(s * nslice, nslice)])
        plsc.subcore_barrier()

        pltpu.sync_copy(sh_winv, winv)
        cp_batch.wait()
        cp_edges.wait()
        lane = lax.iota(jnp.int32, _L)
        tail = ebuf_c[pl.ds(tail_base, _L)]
        ebuf_c[pl.ds(tail_base, _L)] = jnp.where(lane < tail_valid, tail, 0)

        @plsc.parallel_loop(0, g_main // 4, unroll=4)
        def _(i4):
            for k in range(4):
                off = (i4 * 4 + k) * _L
                idx = ebuf_c[pl.ds(off, _L)]
                geb[pl.ds(off, _L)] = plsc.load_gather(batch_l, [idx])
                web[pl.ds(off, _L)] = plsc.load_gather(winv, [idx])

        @plsc.parallel_loop(g_main, g_iters)
        def _(i):
            idx = ebuf_c[pl.ds(i * _L, _L)]
            geb[pl.ds(i * _L, _L)] = plsc.load_gather(batch_l, [idx])
            web[pl.ds(i * _L, _L)] = plsc.load_gather(winv, [idx])

        pltpu.sync_copy(geb.at[pl.ds(0, epw)], ge_hbm.at[pl.ds(w * epw, epw)])
        pltpu.sync_copy(web.at[pl.ds(0, epw)], we_hbm.at[pl.ds(w * epw, epw)])

    return sc_prep


def _mm(a, b):
    return lax.dot_general(a, b, (((1,), (0,)), ((), ())),
                           preferred_element_type=jnp.float32,
                           precision=lax.Precision.HIGHEST)


def _mm_fast(a, b):
    return lax.dot_general(a, b, (((1,), (0,)), ((), ())),
                           preferred_element_type=jnp.float32)


def _make_tc_nodeagg(N, B, DIM, n_blk, nb):
    def body(bt_ref, x_ref, xsum_ref, cnt_ref, acc_v, cnt):
        i = pl.program_id(0)

        @pl.when(i == 0)
        def _():
            acc_v[...] = jnp.zeros_like(acc_v)
            cnt[...] = jnp.zeros_like(cnt)

        bt = bt_ref[0]
        niota = lax.broadcasted_iota(jnp.int32, (B, nb), 0)
        onehot_v = jnp.where(bt == niota, 1.0, 0.0)
        acc_v[...] = acc_v[...] + _mm_fast(onehot_v.astype(jnp.bfloat16),
                                           x_ref[...].astype(jnp.bfloat16))
        cnt[...] = cnt[...] + jnp.sum(onehot_v, axis=1, keepdims=True)

        @pl.when(i == n_blk - 1)
        def _():
            xsum_ref[...] = acc_v[...]
            cnt_ref[...] = cnt[...]

    full2 = lambda i: (0, 0)
    return pl.pallas_call(
        body,
        grid=(n_blk,),
        in_specs=[
            pl.BlockSpec((1, 1, nb), lambda i: (i, 0, 0)),
            pl.BlockSpec((nb, DIM), lambda i: (i, 0)),
        ],
        out_specs=[
            pl.BlockSpec((B, DIM), full2),
            pl.BlockSpec((B, 128), full2),
        ],
        out_shape=[
            jax.ShapeDtypeStruct((B, DIM), jnp.float32),
            jax.ShapeDtypeStruct((B, 128), jnp.float32),
        ],
        scratch_shapes=[
            pltpu.VMEM((B, DIM), jnp.float32),
            pltpu.VMEM((B, 128), jnp.float32),
        ],
        compiler_params=pltpu.CompilerParams(
            dimension_semantics=("arbitrary",)),
    )


def _make_tc_edge(E, B, DIM, n_blk, eb):
    def body(ge_ref, we_ref, ea_ref, xsum_ref, cnt_ref, u_ref,
             w1a_ref, w1b_ref, w1c_ref, b1_ref, w2_ref, b2_ref,
             out_ref, acc_e):
        i = pl.program_id(0)

        @pl.when(i == 0)
        def _():
            acc_e[...] = jnp.zeros_like(acc_e)

        ge = ge_ref[0]
        we = we_ref[0]
        giota = lax.broadcasted_iota(jnp.int32, (B, eb), 0)
        onehot_e = jnp.where(ge == giota, jnp.broadcast_to(we, (B, eb)), 0.0)
        acc_e[...] = acc_e[...] + _mm_fast(onehot_e.astype(jnp.bfloat16),
                                           ea_ref[...].astype(jnp.bfloat16))

        @pl.when(i == n_blk - 1)
        def _():
            n = jnp.maximum(cnt_ref[:, 0:1], 1.0)
            ue = acc_e[...] / n
            uv = xsum_ref[...] / n
            h = (_mm(ue, w1a_ref[...]) + _mm(uv, w1b_ref[...])
                 + _mm(u_ref[...], w1c_ref[...]) + b1_ref[...])
            h = jnp.maximum(h, 0.0)
            out_ref[...] = _mm(h, w2_ref[...]) + b2_ref[...]

    full2 = lambda i: (0, 0)
    return pl.pallas_call(
        body,
        grid=(n_blk,),
        in_specs=[
            pl.BlockSpec((1, 1, eb), lambda i: (i, 0, 0)),
            pl.BlockSpec((1, 1, eb), lambda i: (i, 0, 0)),
            pl.BlockSpec((eb, DIM), lambda i: (i, 0)),
            pl.BlockSpec((B, DIM), full2),
            pl.BlockSpec((B, 128), full2),
            pl.BlockSpec((B, DIM), full2),
            pl.BlockSpec((DIM, DIM), full2),
            pl.BlockSpec((DIM, DIM), full2),
            pl.BlockSpec((DIM, DIM), full2),
            pl.BlockSpec((1, DIM), full2),
            pl.BlockSpec((DIM, DIM), full2),
            pl.BlockSpec((1, DIM), full2),
        ],
        out_specs=pl.BlockSpec((B, DIM), full2),
        out_shape=jax.ShapeDtypeStruct((B, DIM), jnp.float32),
        scratch_shapes=[
            pltpu.VMEM((B, DIM), jnp.float32),
        ],
        compiler_params=pltpu.CompilerParams(
            dimension_semantics=("arbitrary",)),
    )


def kernel(x, edge_index, edge_attr, u, batch, W1, b1, W2, b2):
    N, DIM = x.shape
    E = edge_attr.shape[0]
    B = u.shape[0]

    n_eblk = 20
    eb = E // n_eblk
    n_nblk = 10
    nb = N // n_nblk

    esrc = edge_index[0]
    ge, we = _make_sc_prep(E, N)(esrc, batch)
    xsum, cnt = _make_tc_nodeagg(N, B, DIM, n_nblk, nb)(
        batch.reshape(n_nblk, 1, nb), x)

    out = _make_tc_edge(E, B, DIM, n_eblk, eb)(
        ge.reshape(n_eblk, 1, eb), we.reshape(n_eblk, 1, eb), edge_attr,
        xsum, cnt, u,
        W1[0:DIM], W1[DIM:2 * DIM], W1[2 * DIM:3 * DIM],
        b1.reshape(1, DIM), W2, b2.reshape(1, DIM))
    return out


# speedup vs baseline: 1.1575x; 1.0635x over previous
import functools

import jax
import jax.numpy as jnp
from jax import lax
from jax.experimental import pallas as pl
from jax.experimental.pallas import tpu as pltpu
from jax.experimental.pallas import tpu_sc as plsc

_NC = 2
_NS = 16
_NW = _NC * _NS
_L = 16


def _make_sc_prep(E, N):
    ept_h = E // _NS
    epw = E // _NW
    npad = ((N + _NS * _L - 1) // (_NS * _L)) * (_NS * _L)
    nslice = npad // _NS
    g_iters = (epw + _L - 1) // _L
    tail_base = (g_iters - 1) * _L
    tail_valid = epw - tail_base
    g_main = (g_iters - 1) // 4 * 4

    mesh = plsc.VectorSubcoreMesh(core_axis_name="c", subcore_axis_name="s")

    @functools.partial(
        pl.kernel,
        out_type=(
            jax.ShapeDtypeStruct((E,), jnp.int32),
            jax.ShapeDtypeStruct((E,), jnp.float32),
        ),
        mesh=mesh,
        compiler_params=pltpu.CompilerParams(needs_layout_passes=False),
        scratch_types=[
            pltpu.VMEM((ept_h,), jnp.int32),
            pltpu.VMEM((g_iters * _L,), jnp.int32),
            pltpu.VMEM((npad,), jnp.float32),
            pltpu.VMEM((_NS, nslice), jnp.float32),
            pltpu.VMEM((nslice,), jnp.float32),
            pltpu.VMEM((npad,), jnp.float32),
            pltpu.VMEM((N,), jnp.int32),
            pltpu.VMEM((g_iters * _L,), jnp.int32),
            pltpu.VMEM((g_iters * _L,), jnp.float32),
            pltpu.VMEM_SHARED((_NS, npad), jnp.float32),
            pltpu.VMEM_SHARED((npad,), jnp.float32),
            pltpu.SemaphoreType.DMA,
            pltpu.SemaphoreType.DMA,
        ],
    )
    def sc_prep(eidx_hbm, batch_hbm, ge_hbm, we_hbm,
                ebuf, ebuf_c, hist, parts, winv_s, winv, batch_l, geb, web,
                sh_hist, sh_winv, sem_b, sem_e):
        c = lax.axis_index("c")
        s = lax.axis_index("s")
        w = c * _NS + s
        esrc_hbm = eidx_hbm.at[pl.ds(0, E)]

        cp_batch = pltpu.async_copy(batch_hbm, batch_l, sem_b)
        cp_edges = pltpu.async_copy(esrc_hbm.at[pl.ds(w * epw, epw)],
                                    ebuf_c.at[pl.ds(0, epw)], sem_e)

        @plsc.parallel_loop(0, npad // _L)
        def _(i):
            hist[pl.ds(i * _L, _L)] = jnp.zeros((_L,), jnp.float32)

        pltpu.sync_copy(esrc_hbm.at[pl.ds(s * ept_h, ept_h)], ebuf)
        ones = jnp.ones((_L,), jnp.float32)

        def hist_body(i, _):
            base = i * (5 * _L)
            for k in range(5):
                idx = ebuf[pl.ds(base + k * _L, _L)]
                plsc.addupdate_scatter(hist, [idx], ones)
            return 0
        lax.fori_loop(0, ept_h // (5 * _L), hist_body, 0)

        pltpu.sync_copy(hist, sh_hist.at[s])
        plsc.subcore_barrier()

        pltpu.sync_copy(sh_hist.at[:, pl.ds(s * nslice, nslice)], parts)

        @plsc.parallel_loop(0, nslice // _L)
        def _(j):
            acc = jnp.zeros((_L,), jnp.float32)
            for t in range(_NS):
                acc = acc + parts[t, pl.ds(j * _L, _L)]
            winv_s[pl.ds(j * _L, _L)] = 1.0 / jnp.maximum(acc, 1.0)

        pltpu.sync_copy(winv_s, sh_winv.at[pl.ds(s * nslice, nslice)])
        plsc.subcore_barrier()

        pltpu.sync_copy(sh_winv, winv)
        cp_batch.wait()
        cp_edges.wait()
        lane = lax.iota(jnp.int32, _L)
        tail = ebuf_c[pl.ds(tail_base, _L)]
        ebuf_c[pl.ds(tail_base, _L)] = jnp.where(lane < tail_valid, tail, 0)

        @plsc.parallel_loop(0, g_main // 4, unroll=4)
        def _(i4):
            for k in range(4):
                off = (i4 * 4 + k) * _L
                idx = ebuf_c[pl.ds(off, _L)]
                geb[pl.ds(off, _L)] = plsc.load_gather(batch_l, [idx])
                web[pl.ds(off, _L)] = plsc.load_gather(winv, [idx])

        @plsc.parallel_loop(g_main, g_iters)
        def _(i):
            idx = ebuf_c[pl.ds(i * _L, _L)]
            geb[pl.ds(i * _L, _L)] = plsc.load_gather(batch_l, [idx])
            web[pl.ds(i * _L, _L)] = plsc.load_gather(winv, [idx])

        pltpu.sync_copy(geb.at[pl.ds(0, epw)], ge_hbm.at[pl.ds(w * epw, epw)])
        pltpu.sync_copy(web.at[pl.ds(0, epw)], we_hbm.at[pl.ds(w * epw, epw)])

    return sc_prep


def _mm(a, b):
    return lax.dot_general(a, b, (((1,), (0,)), ((), ())),
                           preferred_element_type=jnp.float32,
                           precision=lax.Precision.HIGHEST)


def _mm_fast(a, b):
    return lax.dot_general(a, b, (((1,), (0,)), ((), ())),
                           preferred_element_type=jnp.float32)


def _make_tc_nodeagg(N, B, DIM, n_blk, nb):
    def body(bt_ref, x_ref, xsum_ref, cnt_ref, acc_v, cnt):
        i = pl.program_id(0)

        @pl.when(i == 0)
        def _():
            acc_v[...] = jnp.zeros_like(acc_v)
            cnt[...] = jnp.zeros_like(cnt)

        bt = bt_ref[0]
        niota = lax.broadcasted_iota(jnp.int32, (B, nb), 0)
        onehot_v = jnp.where(bt == niota, 1.0, 0.0)
        acc_v[...] = acc_v[...] + _mm_fast(onehot_v.astype(jnp.bfloat16),
                                           x_ref[...].astype(jnp.bfloat16))
        cnt[...] = cnt[...] + jnp.sum(onehot_v, axis=1, keepdims=True)

        @pl.when(i == n_blk - 1)
        def _():
            xsum_ref[...] = acc_v[...]
            cnt_ref[...] = cnt[...]

    full2 = lambda i: (0, 0)
    return pl.pallas_call(
        body,
        grid=(n_blk,),
        in_specs=[
            pl.BlockSpec((1, 1, nb), lambda i: (i, 0, 0)),
            pl.BlockSpec((nb, DIM), lambda i: (i, 0)),
        ],
        out_specs=[
            pl.BlockSpec((B, DIM), full2),
            pl.BlockSpec((B, 128), full2),
        ],
        out_shape=[
            jax.ShapeDtypeStruct((B, DIM), jnp.float32),
            jax.ShapeDtypeStruct((B, 128), jnp.float32),
        ],
        scratch_shapes=[
            pltpu.VMEM((B, DIM), jnp.float32),
            pltpu.VMEM((B, 128), jnp.float32),
        ],
        compiler_params=pltpu.CompilerParams(
            dimension_semantics=("arbitrary",)),
    )


def _make_tc_edge(E, B, DIM, n_blk, eb):
    def body(ge_ref, we_ref, ea_ref, xsum_ref, cnt_ref, u_ref,
             w1_ref, b1_ref, w2_ref, b2_ref,
             out_ref, acc_e):
        i = pl.program_id(0)

        @pl.when(i == 0)
        def _():
            acc_e[...] = jnp.zeros_like(acc_e)

        ge = ge_ref[0]
        we = we_ref[0]
        giota = lax.broadcasted_iota(jnp.int32, (B, eb), 0)
        onehot_e = jnp.where(ge == giota, jnp.broadcast_to(we, (B, eb)), 0.0)
        acc_e[...] = acc_e[...] + _mm_fast(onehot_e.astype(jnp.bfloat16),
                                           ea_ref[...].astype(jnp.bfloat16))

        @pl.when(i == n_blk - 1)
        def _():
            n = jnp.maximum(cnt_ref[:, 0:1], 1.0)
            ue = acc_e[...] / n
            uv = xsum_ref[...] / n
            h = (_mm(ue, w1_ref[0:DIM, :]) + _mm(uv, w1_ref[DIM:2 * DIM, :])
                 + _mm(u_ref[...], w1_ref[2 * DIM:3 * DIM, :]) + b1_ref[...])
            h = jnp.maximum(h, 0.0)
            out_ref[...] = _mm(h, w2_ref[...]) + b2_ref[...]

    full2 = lambda i: (0, 0)
    return pl.pallas_call(
        body,
        grid=(n_blk,),
        in_specs=[
            pl.BlockSpec((1, 1, eb), lambda i: (i, 0, 0)),
            pl.BlockSpec((1, 1, eb), lambda i: (i, 0, 0)),
            pl.BlockSpec((eb, DIM), lambda i: (i, 0)),
            pl.BlockSpec((B, DIM), full2),
            pl.BlockSpec((B, 128), full2),
            pl.BlockSpec((B, DIM), full2),
            pl.BlockSpec((3 * DIM, DIM), full2),
            pl.BlockSpec((1, DIM), full2),
            pl.BlockSpec((DIM, DIM), full2),
            pl.BlockSpec((1, DIM), full2),
        ],
        out_specs=pl.BlockSpec((B, DIM), full2),
        out_shape=jax.ShapeDtypeStruct((B, DIM), jnp.float32),
        scratch_shapes=[
            pltpu.VMEM((B, DIM), jnp.float32),
        ],
        compiler_params=pltpu.CompilerParams(
            dimension_semantics=("arbitrary",)),
    )


def kernel(x, edge_index, edge_attr, u, batch, W1, b1, W2, b2):
    N, DIM = x.shape
    E = edge_attr.shape[0]
    B = u.shape[0]

    n_eblk = 25
    eb = E // n_eblk
    n_nblk = 10
    nb = N // n_nblk

    ge, we = _make_sc_prep(E, N)(edge_index.reshape(2 * E), batch)
    xsum, cnt = _make_tc_nodeagg(N, B, DIM, n_nblk, nb)(
        batch.reshape(n_nblk, 1, nb), x)

    out = _make_tc_edge(E, B, DIM, n_eblk, eb)(
        ge.reshape(n_eblk, 1, eb), we.reshape(n_eblk, 1, eb), edge_attr,
        xsum, cnt, u,
        W1, b1.reshape(1, DIM), W2, b2.reshape(1, DIM))
    return out


# speedup vs baseline: 1.1688x; 1.0098x over previous
import functools

import jax
import jax.numpy as jnp
from jax import lax
from jax.experimental import pallas as pl
from jax.experimental.pallas import tpu as pltpu
from jax.experimental.pallas import tpu_sc as plsc

_NC = 2
_NS = 16
_NW = _NC * _NS
_L = 16


def _make_sc_prep(E, N):
    ept_h = (E // (_NS * 128)) * 128
    epw = (E // (_NW * 64)) * 64
    main_base = _NW * epw
    rem = E - main_base
    rem_h = rem // _NS
    npad = ((N + _NS * _L - 1) // (_NS * _L)) * (_NS * _L)
    nslice = npad // _NS
    g_main = epw // _L

    mesh = plsc.VectorSubcoreMesh(core_axis_name="c", subcore_axis_name="s")

    @functools.partial(
        pl.kernel,
        out_type=jax.ShapeDtypeStruct((E,), jnp.float32),
        mesh=mesh,
        compiler_params=pltpu.CompilerParams(needs_layout_passes=False),
        scratch_types=[
            pltpu.VMEM((ept_h + rem,), jnp.int32),
            pltpu.VMEM((epw + rem,), jnp.int32),
            pltpu.VMEM((npad,), jnp.float32),
            pltpu.VMEM((_NS, nslice), jnp.float32),
            pltpu.VMEM((nslice,), jnp.float32),
            pltpu.VMEM((npad,), jnp.float32),
            pltpu.VMEM((N,), jnp.int32),
            pltpu.VMEM((epw + rem,), jnp.float32),
            pltpu.VMEM_SHARED((_NS, npad), jnp.float32),
            pltpu.VMEM_SHARED((npad,), jnp.float32),
            pltpu.SemaphoreType.DMA,
            pltpu.SemaphoreType.DMA,
        ],
    )
    def sc_prep(eidx_hbm, batch_hbm, combo_hbm,
                ebuf, ebuf_c, hist, parts, winv_s, winv, batch_l, cbuf,
                sh_hist, sh_winv, sem_b, sem_e):
        c = lax.axis_index("c")
        s = lax.axis_index("s")
        w = c * _NS + s
        esrc = eidx_hbm.at[0]

        cp_batch = pltpu.async_copy(batch_hbm, batch_l, sem_b)
        cp_edges = pltpu.async_copy(esrc.at[pl.ds(w * epw, epw)],
                                    ebuf_c.at[pl.ds(0, epw)], sem_e)

        @plsc.parallel_loop(0, npad // _L)
        def _(i):
            hist[pl.ds(i * _L, _L)] = jnp.zeros((_L,), jnp.float32)

        pltpu.sync_copy(esrc.at[pl.ds(s * ept_h, ept_h)],
                        ebuf.at[pl.ds(0, ept_h)])
        pltpu.sync_copy(esrc.at[pl.ds(main_base, rem)],
                        ebuf.at[pl.ds(ept_h, rem)])
        ones = jnp.ones((_L,), jnp.float32)

        def hist_body(i, _):
            base = i * (4 * _L)
            for k in range(4):
                idx = ebuf[pl.ds(base + k * _L, _L)]
                plsc.addupdate_scatter(hist, [idx], ones)
            return 0
        lax.fori_loop(0, ept_h // (4 * _L), hist_body, 0)
        plsc.addupdate_scatter(hist, [ebuf[pl.ds(ept_h + s * rem_h, _L)]],
                               ones)

        pltpu.sync_copy(hist, sh_hist.at[s])
        plsc.subcore_barrier()

        pltpu.sync_copy(sh_hist.at[:, pl.ds(s * nslice, nslice)], parts)

        @plsc.parallel_loop(0, nslice // _L)
        def _(j):
            acc = jnp.zeros((_L,), jnp.float32)
            for t in range(_NS):
                acc = acc + parts[t, pl.ds(j * _L, _L)]
            winv_s[pl.ds(j * _L, _L)] = 1.0 / jnp.maximum(acc, 1.0)

        pltpu.sync_copy(winv_s, sh_winv.at[pl.ds(s * nslice, nslice)])
        plsc.subcore_barrier()

        pltpu.sync_copy(sh_winv, winv)

        @pl.when(w == 0)
        def _():
            pltpu.sync_copy(esrc.at[pl.ds(main_base, rem)],
                            ebuf_c.at[pl.ds(epw, rem)])

        cp_batch.wait()
        cp_edges.wait()

        def gat_one(i):
            idx = ebuf_c[pl.ds(i * _L, _L)]
            gi = plsc.load_gather(batch_l, [idx])
            wv = plsc.load_gather(winv, [idx])
            cbuf[pl.ds(i * _L, _L)] = gi.astype(jnp.float32) + 0.5 * wv

        @plsc.parallel_loop(0, g_main // 4, unroll=4)
        def _(i4):
            for k in range(4):
                gat_one(i4 * 4 + k)

        pltpu.sync_copy(cbuf.at[pl.ds(0, epw)],
                        combo_hbm.at[pl.ds(w * epw, epw)])

        @pl.when(w == 0)
        def _():
            @plsc.parallel_loop(g_main, g_main + rem // _L)
            def _(i):
                gat_one(i)
            pltpu.sync_copy(cbuf.at[pl.ds(epw, rem)],
                            combo_hbm.at[pl.ds(main_base, rem)])

    return sc_prep


def _mm(a, b):
    return lax.dot_general(a, b, (((1,), (0,)), ((), ())),
                           preferred_element_type=jnp.float32,
                           precision=lax.Precision.HIGHEST)


def _mm_fast(a, b):
    return lax.dot_general(a, b, (((1,), (0,)), ((), ())),
                           preferred_element_type=jnp.float32)


def _make_tc_nodeagg(N, B, DIM, n_blk, nb):
    def body(bt_ref, x_ref, xsum_ref, cnt_ref, acc_v, cnt):
        i = pl.program_id(0)

        @pl.when(i == 0)
        def _():
            acc_v[...] = jnp.zeros_like(acc_v)
            cnt[...] = jnp.zeros_like(cnt)

        bt = bt_ref[0]
        niota = lax.broadcasted_iota(jnp.int32, (B, nb), 0)
        onehot_v = jnp.where(bt == niota, 1.0, 0.0)
        acc_v[...] = acc_v[...] + _mm_fast(onehot_v.astype(jnp.bfloat16),
                                           x_ref[...].astype(jnp.bfloat16))
        cnt[...] = cnt[...] + jnp.sum(onehot_v, axis=1, keepdims=True)

        @pl.when(i == n_blk - 1)
        def _():
            xsum_ref[...] = acc_v[...]
            cnt_ref[...] = cnt[...]

    full2 = lambda i: (0, 0)
    return pl.pallas_call(
        body,
        grid=(n_blk,),
        in_specs=[
            pl.BlockSpec((1, 1, nb), lambda i: (i, 0, 0)),
            pl.BlockSpec((nb, DIM), lambda i: (i, 0)),
        ],
        out_specs=[
            pl.BlockSpec((B, DIM), full2),
            pl.BlockSpec((B, 128), full2),
        ],
        out_shape=[
            jax.ShapeDtypeStruct((B, DIM), jnp.float32),
            jax.ShapeDtypeStruct((B, 128), jnp.float32),
        ],
        scratch_shapes=[
            pltpu.VMEM((B, DIM), jnp.float32),
            pltpu.VMEM((B, 128), jnp.float32),
        ],
        compiler_params=pltpu.CompilerParams(
            dimension_semantics=("arbitrary",)),
    )


def _make_tc_edge(E, B, DIM, n_blk, eb):
    def body(cb_ref, ea_ref, xsum_ref, cnt_ref, u_ref,
             w1_ref, b1_ref, w2_ref, b2_ref,
             out_ref, acc_e):
        i = pl.program_id(0)

        @pl.when(i == 0)
        def _():
            acc_e[...] = jnp.zeros_like(acc_e)

        v = cb_ref[0]
        gef = jnp.floor(v)
        we = (v - gef) * 2.0
        ge = gef.astype(jnp.int32)
        giota = lax.broadcasted_iota(jnp.int32, (B, eb), 0)
        onehot_e = jnp.where(ge == giota, jnp.broadcast_to(we, (B, eb)), 0.0)
        acc_e[...] = acc_e[...] + _mm_fast(onehot_e.astype(jnp.bfloat16),
                                           ea_ref[...].astype(jnp.bfloat16))

        @pl.when(i == n_blk - 1)
        def _():
            n = jnp.maximum(cnt_ref[:, 0:1], 1.0)
            ue = acc_e[...] / n
            uv = xsum_ref[...] / n
            h = (_mm(ue, w1_ref[0:DIM, :]) + _mm(uv, w1_ref[DIM:2 * DIM, :])
                 + _mm(u_ref[...], w1_ref[2 * DIM:3 * DIM, :]) + b1_ref[...])
            h = jnp.maximum(h, 0.0)
            out_ref[...] = _mm(h, w2_ref[...]) + b2_ref[...]

    full2 = lambda i: (0, 0)
    return pl.pallas_call(
        body,
        grid=(n_blk,),
        in_specs=[
            pl.BlockSpec((1, 1, eb), lambda i: (i, 0, 0)),
            pl.BlockSpec((eb, DIM), lambda i: (i, 0)),
            pl.BlockSpec((B, DIM), full2),
            pl.BlockSpec((B, 128), full2),
            pl.BlockSpec((B, DIM), full2),
            pl.BlockSpec((3 * DIM, DIM), full2),
            pl.BlockSpec((1, DIM), full2),
            pl.BlockSpec((DIM, DIM), full2),
            pl.BlockSpec((1, DIM), full2),
        ],
        out_specs=pl.BlockSpec((B, DIM), full2),
        out_shape=jax.ShapeDtypeStruct((B, DIM), jnp.float32),
        scratch_shapes=[
            pltpu.VMEM((B, DIM), jnp.float32),
        ],
        compiler_params=pltpu.CompilerParams(
            dimension_semantics=("arbitrary",)),
    )


def kernel(x, edge_index, edge_attr, u, batch, W1, b1, W2, b2):
    N, DIM = x.shape
    E = edge_attr.shape[0]
    B = u.shape[0]

    n_eblk = 25
    eb = E // n_eblk
    n_nblk = 10
    nb = N // n_nblk

    combo = _make_sc_prep(E, N)(edge_index, batch)
    xsum, cnt = _make_tc_nodeagg(N, B, DIM, n_nblk, nb)(
        batch.reshape(n_nblk, 1, nb), x)

    out = _make_tc_edge(E, B, DIM, n_eblk, eb)(
        combo.reshape(n_eblk, 1, eb), edge_attr,
        xsum, cnt, u,
        W1, b1.reshape(1, DIM), W2, b2.reshape(1, DIM))
    return out


# speedup vs baseline: 1.1792x; 1.0089x over previous
import functools

import jax
import jax.numpy as jnp
from jax import lax
from jax.experimental import pallas as pl
from jax.experimental.pallas import tpu as pltpu
from jax.experimental.pallas import tpu_sc as plsc

_NC = 2
_NS = 16
_NW = _NC * _NS
_L = 16


def _make_sc_prep(E, N):
    ept_h = (E // (_NS * 128)) * 128
    epw = (E // (_NW * 64)) * 64
    main_base = _NW * epw
    rem = E - main_base
    rem_h = rem // _NS
    npad = ((N + _NS * _L - 1) // (_NS * _L)) * (_NS * _L)
    nslice = npad // _NS
    g_main = epw // _L

    mesh = plsc.VectorSubcoreMesh(core_axis_name="c", subcore_axis_name="s")

    @functools.partial(
        pl.kernel,
        out_type=jax.ShapeDtypeStruct((E,), jnp.float32),
        mesh=mesh,
        compiler_params=pltpu.CompilerParams(needs_layout_passes=False),
        scratch_types=[
            pltpu.VMEM((ept_h + rem,), jnp.int32),
            pltpu.VMEM((epw + rem,), jnp.int32),
            pltpu.VMEM((npad,), jnp.float32),
            pltpu.VMEM((_NS, nslice), jnp.float32),
            pltpu.VMEM((nslice,), jnp.float32),
            pltpu.VMEM((npad,), jnp.float32),
            pltpu.VMEM((npad,), jnp.int32),
            pltpu.VMEM((epw + rem,), jnp.float32),
            pltpu.VMEM_SHARED((_NS, npad), jnp.float32),
            pltpu.VMEM_SHARED((npad,), jnp.float32),
            pltpu.SemaphoreType.DMA,
            pltpu.SemaphoreType.DMA,
        ],
    )
    def sc_prep(eidx_hbm, batch_hbm, combo_hbm,
                ebuf, ebuf_c, hist, parts, winv_s, winv, batch_l, cbuf,
                sh_hist, sh_winv, sem_b, sem_e):
        c = lax.axis_index("c")
        s = lax.axis_index("s")
        w = c * _NS + s
        esrc = eidx_hbm.at[0]

        cp_batch = pltpu.async_copy(batch_hbm, batch_l.at[pl.ds(0, N)], sem_b)
        cp_edges = pltpu.async_copy(esrc.at[pl.ds(w * epw, epw)],
                                    ebuf_c.at[pl.ds(0, epw)], sem_e)

        @plsc.parallel_loop(0, npad // _L)
        def _(i):
            hist[pl.ds(i * _L, _L)] = jnp.zeros((_L,), jnp.float32)

        pltpu.sync_copy(esrc.at[pl.ds(s * ept_h, ept_h)],
                        ebuf.at[pl.ds(0, ept_h)])
        pltpu.sync_copy(esrc.at[pl.ds(main_base, rem)],
                        ebuf.at[pl.ds(ept_h, rem)])
        ones = jnp.ones((_L,), jnp.float32)

        def hist_body(i, _):
            base = i * (4 * _L)
            for k in range(4):
                idx = ebuf[pl.ds(base + k * _L, _L)]
                plsc.addupdate_scatter(hist, [idx], ones)
            return 0
        lax.fori_loop(0, ept_h // (4 * _L), hist_body, 0)
        plsc.addupdate_scatter(hist, [ebuf[pl.ds(ept_h + s * rem_h, _L)]],
                               ones)

        pltpu.sync_copy(hist, sh_hist.at[s])
        plsc.subcore_barrier()

        pltpu.sync_copy(sh_hist.at[:, pl.ds(s * nslice, nslice)], parts)
        cp_batch.wait()

        @plsc.parallel_loop(0, nslice // _L)
        def _(j):
            acc = jnp.zeros((_L,), jnp.float32)
            for t in range(_NS):
                acc = acc + parts[t, pl.ds(j * _L, _L)]
            bt = batch_l[pl.ds(s * nslice + j * _L, _L)]
            winv_s[pl.ds(j * _L, _L)] = (bt.astype(jnp.float32)
                                         + 0.5 / jnp.maximum(acc, 1.0))

        pltpu.sync_copy(winv_s, sh_winv.at[pl.ds(s * nslice, nslice)])
        plsc.subcore_barrier()

        pltpu.sync_copy(sh_winv, winv)

        @pl.when(w == 0)
        def _():
            pltpu.sync_copy(esrc.at[pl.ds(main_base, rem)],
                            ebuf_c.at[pl.ds(epw, rem)])

        cp_edges.wait()

        def gat_one(i):
            idx = ebuf_c[pl.ds(i * _L, _L)]
            cbuf[pl.ds(i * _L, _L)] = plsc.load_gather(winv, [idx])

        @plsc.parallel_loop(0, g_main // 4, unroll=4)
        def _(i4):
            for k in range(4):
                gat_one(i4 * 4 + k)

        pltpu.sync_copy(cbuf.at[pl.ds(0, epw)],
                        combo_hbm.at[pl.ds(w * epw, epw)])

        @pl.when(w == 0)
        def _():
            @plsc.parallel_loop(g_main, g_main + rem // _L)
            def _(i):
                gat_one(i)
            pltpu.sync_copy(cbuf.at[pl.ds(epw, rem)],
                            combo_hbm.at[pl.ds(main_base, rem)])

    return sc_prep


def _mm(a, b):
    return lax.dot_general(a, b, (((1,), (0,)), ((), ())),
                           preferred_element_type=jnp.float32,
                           precision=lax.Precision.HIGHEST)


def _mm_fast(a, b):
    return lax.dot_general(a, b, (((1,), (0,)), ((), ())),
                           preferred_element_type=jnp.float32)


def _make_tc_nodeagg(N, B, DIM, n_blk, nb):
    def body(bt_ref, x_ref, xsum_ref, cnt_ref, acc_v, cnt):
        i = pl.program_id(0)

        @pl.when(i == 0)
        def _():
            acc_v[...] = jnp.zeros_like(acc_v)
            cnt[...] = jnp.zeros_like(cnt)

        bt = bt_ref[0]
        niota = lax.broadcasted_iota(jnp.int32, (B, nb), 0)
        onehot_v = jnp.where(bt == niota, 1.0, 0.0)
        acc_v[...] = acc_v[...] + _mm_fast(onehot_v.astype(jnp.bfloat16),
                                           x_ref[...].astype(jnp.bfloat16))
        cnt[...] = cnt[...] + jnp.sum(onehot_v, axis=1, keepdims=True)

        @pl.when(i == n_blk - 1)
        def _():
            xsum_ref[...] = acc_v[...]
            cnt_ref[...] = cnt[...]

    full2 = lambda i: (0, 0)
    return pl.pallas_call(
        body,
        grid=(n_blk,),
        in_specs=[
            pl.BlockSpec((1, 1, nb), lambda i: (i, 0, 0)),
            pl.BlockSpec((nb, DIM), lambda i: (i, 0)),
        ],
        out_specs=[
            pl.BlockSpec((B, DIM), full2),
            pl.BlockSpec((B, 128), full2),
        ],
        out_shape=[
            jax.ShapeDtypeStruct((B, DIM), jnp.float32),
            jax.ShapeDtypeStruct((B, 128), jnp.float32),
        ],
        scratch_shapes=[
            pltpu.VMEM((B, DIM), jnp.float32),
            pltpu.VMEM((B, 128), jnp.float32),
        ],
        compiler_params=pltpu.CompilerParams(
            dimension_semantics=("arbitrary",)),
    )


def _make_tc_edge(E, B, DIM, n_blk, eb):
    def body(cb_ref, ea_ref, xsum_ref, cnt_ref, u_ref,
             w1_ref, b1_ref, w2_ref, b2_ref,
             out_ref, acc_e):
        i = pl.program_id(0)

        @pl.when(i == 0)
        def _():
            acc_e[...] = jnp.zeros_like(acc_e)

        v = cb_ref[0]
        gef = jnp.floor(v)
        we = (v - gef) * 2.0
        ge = gef.astype(jnp.int32)
        giota = lax.broadcasted_iota(jnp.int32, (B, eb), 0)
        onehot_e = jnp.where(ge == giota, jnp.broadcast_to(we, (B, eb)), 0.0)
        acc_e[...] = acc_e[...] + _mm_fast(onehot_e.astype(jnp.bfloat16),
                                           ea_ref[...].astype(jnp.bfloat16))

        @pl.when(i == n_blk - 1)
        def _():
            n = jnp.maximum(cnt_ref[:, 0:1], 1.0)
            ue = acc_e[...] / n
            uv = xsum_ref[...] / n
            h = (_mm(ue, w1_ref[0:DIM, :]) + _mm(uv, w1_ref[DIM:2 * DIM, :])
                 + _mm(u_ref[...], w1_ref[2 * DIM:3 * DIM, :]) + b1_ref[...])
            h = jnp.maximum(h, 0.0)
            out_ref[...] = _mm(h, w2_ref[...]) + b2_ref[...]

    full2 = lambda i: (0, 0)
    return pl.pallas_call(
        body,
        grid=(n_blk,),
        in_specs=[
            pl.BlockSpec((1, 1, eb), lambda i: (i, 0, 0)),
            pl.BlockSpec((eb, DIM), lambda i: (i, 0)),
            pl.BlockSpec((B, DIM), full2),
            pl.BlockSpec((B, 128), full2),
            pl.BlockSpec((B, DIM), full2),
            pl.BlockSpec((3 * DIM, DIM), full2),
            pl.BlockSpec((1, DIM), full2),
            pl.BlockSpec((DIM, DIM), full2),
            pl.BlockSpec((1, DIM), full2),
        ],
        out_specs=pl.BlockSpec((B, DIM), full2),
        out_shape=jax.ShapeDtypeStruct((B, DIM), jnp.float32),
        scratch_shapes=[
            pltpu.VMEM((B, DIM), jnp.float32),
        ],
        compiler_params=pltpu.CompilerParams(
            dimension_semantics=("arbitrary",)),
    )


def kernel(x, edge_index, edge_attr, u, batch, W1, b1, W2, b2):
    N, DIM = x.shape
    E = edge_attr.shape[0]
    B = u.shape[0]

    n_eblk = 25
    eb = E // n_eblk
    n_nblk = 10
    nb = N // n_nblk

    combo = _make_sc_prep(E, N)(edge_index, batch)
    xsum, cnt = _make_tc_nodeagg(N, B, DIM, n_nblk, nb)(
        batch.reshape(n_nblk, 1, nb), x)

    out = _make_tc_edge(E, B, DIM, n_eblk, eb)(
        combo.reshape(n_eblk, 1, eb), edge_attr,
        xsum, cnt, u,
        W1, b1.reshape(1, DIM), W2, b2.reshape(1, DIM))
    return out
